# baseline (speedup 1.0000x reference)
import functools
import math

import jax
import jax.numpy as jnp
from jax.experimental import pallas as pl
from jax.experimental.pallas import tpu as pltpu

_EPS = 1e-06
_TEMPERATURE = 0.07
_INV_TEMP = 1.0 / max(_TEMPERATURE, _EPS)
_P_SCALE = 14.0
_EXP2_C = math.log2(math.e) / _P_SCALE


def _lse_body(t0_ref, t1_ref, proto_ref, out_ref, tokn, m_s, s_s):
    b = pl.program_id(0)
    j = pl.program_id(2)
    nj = pl.num_programs(2)
    bm = tokn.shape[0]
    bn = proto_ref.shape[2]

    @pl.when(j == 0)
    def _init():
        def _norm(x):
            n = jnp.sqrt(jnp.sum(x * x, axis=1, keepdims=True))
            scale = _INV_TEMP / jnp.maximum(n, 1e-12)
            tokn[...] = (x * scale).astype(tokn.dtype)

        @pl.when(b == 0)
        def _():
            _norm(t0_ref[0, 1:, :])

        @pl.when(b == 1)
        def _():
            _norm(t1_ref[0, 1:, :])

        m_s[...] = jnp.full_like(m_s, -jnp.inf)
        s_s[...] = jnp.zeros_like(s_s)

    cw = 1024
    m_parts = []
    s_parts = []
    for c in range(bn // cw):
        raw = jax.lax.dot_general(
            tokn[...], proto_ref[0, :, c * cw:(c + 1) * cw],
            (((1,), (0,)), ((), ())),
            preferred_element_type=jnp.float32,
        )
        m_parts.append(jnp.max(raw, axis=1, keepdims=True))
        s_parts.append(jnp.sum(jnp.exp2(raw * _EXP2_C), axis=1, keepdims=True))
    m_new = functools.reduce(jnp.maximum, m_parts)
    s_new = functools.reduce(jnp.add, s_parts)
    m_s[...] = jnp.maximum(m_s[...], m_new)
    s_s[...] = s_s[...] + s_new

    @pl.when(j == nj - 1)
    def _finish():
        out_ref[0] = m_s[...] * (1.0 / _P_SCALE) - jnp.log(s_s[...])


@functools.partial(jax.jit, static_argnames=("bn",))
def _argmax_logp(t0, t1, proto_t, bn):
    bsz, p1, k = t0.shape
    bm = p1 - 1
    n = proto_t.shape[2]
    grid = (2, bsz, n // bn)
    ni = bsz
    return pl.pallas_call(
        _lse_body,
        grid=grid,
        in_specs=[
            pl.BlockSpec((1, p1, k), lambda b, i, j: ((1 - b) * i + b * (ni - 1), 0, 0)),
            pl.BlockSpec((1, p1, k), lambda b, i, j: (b * i, 0, 0)),
            pl.BlockSpec((1, k, bn), lambda b, i, j: (b, 0, j)),
        ],
        out_specs=pl.BlockSpec((1, bm, 1), lambda b, i, j: (b, i, 0)),
        out_shape=jax.ShapeDtypeStruct((2, bsz * bm, 1), jnp.float32),
        scratch_shapes=[
            pltpu.VMEM((bm, k), proto_t.dtype),
            pltpu.VMEM((bm, 1), jnp.float32),
            pltpu.VMEM((bm, 1), jnp.float32),
        ],
        compiler_params=pltpu.CompilerParams(
            dimension_semantics=("parallel", "parallel", "arbitrary"),
        ),
    )(t0, t1, proto_t)


def kernel(img_tokens_0, img_tokens_1, prototypes, normal_mask):
    proto_t = (prototypes * _P_SCALE).astype(jnp.float8_e4m3fn).transpose(0, 2, 1)
    bn = min(8192, prototypes.shape[1])
    vals = _argmax_logp(img_tokens_0, img_tokens_1, proto_t, bn)[..., 0]

    w = normal_mask.reshape(-1).astype(jnp.float32)
    ce = -(vals * w[None, :]).sum(axis=1) / w.sum()
    return ce.mean()


# speedup vs baseline: 1.0923x; 1.0923x over previous
import functools
import math

import jax
import jax.numpy as jnp
from jax.experimental import pallas as pl
from jax.experimental.pallas import tpu as pltpu

_EPS = 1e-06
_TEMPERATURE = 0.07
_INV_TEMP = 1.0 / max(_TEMPERATURE, _EPS)
_P_SCALE = 14.0
_EXP2_C = math.log2(math.e) / _P_SCALE


def _lse_body(t0_ref, t1_ref, proto_ref, out_ref, tokn, m_s, s_s):
    b = pl.program_id(0)
    j = pl.program_id(2)
    nj = pl.num_programs(2)
    bm = tokn.shape[0]
    bn = proto_ref.shape[2]

    @pl.when(j == 0)
    def _init():
        def _norm(x):
            n = jnp.sqrt(jnp.sum(x * x, axis=1, keepdims=True))
            scale = _INV_TEMP / jnp.maximum(n, 1e-12)
            tokn[...] = (x * scale).astype(tokn.dtype)

        @pl.when(b == 0)
        def _():
            _norm(t0_ref[0, 1:, :])

        @pl.when(b == 1)
        def _():
            _norm(t1_ref[0, 1:, :])

        m_s[...] = jnp.full_like(m_s, -jnp.inf)
        s_s[...] = jnp.zeros_like(s_s)

    raw = jax.lax.dot_general(
        tokn[...], proto_ref[0],
        (((1,), (0,)), ((), ())),
        preferred_element_type=jnp.float32,
    ).astype(jnp.bfloat16)

    m_part = jnp.max(raw, axis=1, keepdims=True).astype(jnp.float32)
    e = jnp.exp2(raw * jnp.bfloat16(_EXP2_C))
    s_part = jnp.sum(e, axis=1, keepdims=True, dtype=jnp.bfloat16)
    m_s[...] = jnp.maximum(m_s[...], m_part)
    s_s[...] = s_s[...] + s_part.astype(jnp.float32)

    @pl.when(j == nj - 1)
    def _finish():
        out_ref[0] = m_s[...] * (1.0 / _P_SCALE) - jnp.log(s_s[...])


@functools.partial(jax.jit, static_argnames=("bn",))
def _argmax_logp(t0, t1, proto_t, bn):
    bsz, p1, k = t0.shape
    bm = p1 - 1
    n = proto_t.shape[2]
    grid = (2, bsz, n // bn)
    ni = bsz
    return pl.pallas_call(
        _lse_body,
        grid=grid,
        in_specs=[
            pl.BlockSpec((1, p1, k), lambda b, i, j: ((1 - b) * i + b * (ni - 1), 0, 0)),
            pl.BlockSpec((1, p1, k), lambda b, i, j: (b * i, 0, 0)),
            pl.BlockSpec((1, k, bn), lambda b, i, j: (b, 0, j)),
        ],
        out_specs=pl.BlockSpec((1, bm, 1), lambda b, i, j: (b, i, 0)),
        out_shape=jax.ShapeDtypeStruct((2, bsz * bm, 1), jnp.float32),
        scratch_shapes=[
            pltpu.VMEM((bm, k), proto_t.dtype),
            pltpu.VMEM((bm, 1), jnp.float32),
            pltpu.VMEM((bm, 1), jnp.float32),
        ],
        compiler_params=pltpu.CompilerParams(
            dimension_semantics=("parallel", "parallel", "arbitrary"),
        ),
    )(t0, t1, proto_t)


def kernel(img_tokens_0, img_tokens_1, prototypes, normal_mask):
    proto_t = (prototypes * _P_SCALE).astype(jnp.float8_e4m3fn).transpose(0, 2, 1)
    bn = min(8192, prototypes.shape[1])
    vals = _argmax_logp(img_tokens_0, img_tokens_1, proto_t, bn)[..., 0]

    w = normal_mask.reshape(-1).astype(jnp.float32)
    ce = -(vals * w[None, :]).sum(axis=1) / w.sum()
    return ce.mean()


# speedup vs baseline: 1.3009x; 1.1909x over previous
import functools
import math

import jax
import jax.numpy as jnp
from jax.experimental import pallas as pl
from jax.experimental.pallas import tpu as pltpu

_EPS = 1e-06
_TEMPERATURE = 0.07
_INV_TEMP = 1.0 / max(_TEMPERATURE, _EPS)
_P_SCALE = 1.0
_EXP2_C = math.log2(math.e) / _P_SCALE


def _lse_body(t0_ref, t1_ref, proto_ref, out_ref, tokn, m_s, s_s):
    b = pl.program_id(0)
    j = pl.program_id(2)
    nj = pl.num_programs(2)
    bm = tokn.shape[0]
    bn = proto_ref.shape[2]

    @pl.when(j == 0)
    def _init():
        def _norm(x):
            n = jnp.sqrt(jnp.sum(x * x, axis=1, keepdims=True))
            scale = _INV_TEMP / jnp.maximum(n, 1e-12)
            tokn[...] = (x * scale).astype(tokn.dtype)

        @pl.when(b == 0)
        def _():
            _norm(t0_ref[0, 1:, :])

        @pl.when(b == 1)
        def _():
            _norm(t1_ref[0, 1:, :])

        m_s[...] = jnp.full_like(m_s, -jnp.inf)
        s_s[...] = jnp.zeros_like(s_s)

    raw = jax.lax.dot_general(
        tokn[...], proto_ref[0],
        (((1,), (1,)), ((), ())),
        preferred_element_type=jnp.float32,
    ).astype(jnp.bfloat16)

    m_part = jnp.max(raw, axis=1, keepdims=True).astype(jnp.float32)
    e = jnp.exp2(raw * jnp.bfloat16(_EXP2_C))
    s_part = jnp.sum(e, axis=1, keepdims=True, dtype=jnp.bfloat16)
    m_s[...] = jnp.maximum(m_s[...], m_part)
    s_s[...] = s_s[...] + s_part.astype(jnp.float32)

    @pl.when(j == nj - 1)
    def _finish():
        out_ref[0] = m_s[...] * (1.0 / _P_SCALE) - jnp.log(s_s[...])


@functools.partial(jax.jit, static_argnames=("bn",))
def _argmax_logp(t0, t1, proto_t, bn):
    bsz, p1, k = t0.shape
    bm = p1 - 1
    n = proto_t.shape[1]
    grid = (2, bsz, n // bn)
    ni = bsz
    return pl.pallas_call(
        _lse_body,
        grid=grid,
        in_specs=[
            pl.BlockSpec((1, p1, k), lambda b, i, j: ((1 - b) * i + b * (ni - 1), 0, 0)),
            pl.BlockSpec((1, p1, k), lambda b, i, j: (b * i, 0, 0)),
            pl.BlockSpec((1, bn, k), lambda b, i, j: (b, j, 0)),
        ],
        out_specs=pl.BlockSpec((1, bm, 1), lambda b, i, j: (b, i, 0)),
        out_shape=jax.ShapeDtypeStruct((2, bsz * bm, 1), jnp.float32),
        scratch_shapes=[
            pltpu.VMEM((bm, k), proto_t.dtype),
            pltpu.VMEM((bm, 1), jnp.float32),
            pltpu.VMEM((bm, 1), jnp.float32),
        ],
        compiler_params=pltpu.CompilerParams(
            dimension_semantics=("parallel", "parallel", "arbitrary"),
        ),
    )(t0, t1, proto_t)


def kernel(img_tokens_0, img_tokens_1, prototypes, normal_mask):
    proto_t = prototypes.astype(jnp.float8_e4m3fn)
    bn = min(8192, prototypes.shape[1])
    vals = _argmax_logp(img_tokens_0, img_tokens_1, proto_t, bn)[..., 0]

    w = normal_mask.reshape(-1).astype(jnp.float32)
    ce = -(vals * w[None, :]).sum(axis=1) / w.sum()
    return ce.mean()


# speedup vs baseline: 2.6491x; 2.0364x over previous
import functools
import math

import jax
import jax.numpy as jnp
from jax.experimental import pallas as pl
from jax.experimental.pallas import tpu as pltpu

_EPS = 1e-06
_TEMPERATURE = 0.07
_INV_TEMP = 1.0 / max(_TEMPERATURE, _EPS)
_P_SCALE = 1.0
_EXP2_C = math.log2(math.e) / _P_SCALE


def _lse_body(t0_ref, t1_ref, proto_ref, out_ref, tokn, m_s, s_s):
    b = pl.program_id(0)
    j = pl.program_id(2)
    nj = pl.num_programs(2)
    bm = tokn.shape[0]
    bn = proto_ref.shape[2]

    @pl.when(j == 0)
    def _init():
        def _norm(x):
            n = jnp.sqrt(jnp.sum(x * x, axis=1, keepdims=True))
            scale = _INV_TEMP / jnp.maximum(n, 1e-12)
            tokn[...] = (x * scale).astype(tokn.dtype)

        @pl.when(b == 0)
        def _():
            _norm(t0_ref[0, 1:, :])

        @pl.when(b == 1)
        def _():
            _norm(t1_ref[0, 1:, :])

        m_s[...] = jnp.full_like(m_s, -jnp.inf)
        s_s[...] = jnp.zeros_like(s_s)

    half = bn // 2
    for c in range(2):
        raw = jax.lax.dot_general(
            tokn[...], proto_ref[0, c * half:(c + 1) * half, :],
            (((1,), (1,)), ((), ())),
            preferred_element_type=jnp.float32,
        ).astype(jnp.bfloat16)

        m_part = jnp.max(raw, axis=1, keepdims=True).astype(jnp.float32)
        e = jnp.exp2(raw * jnp.bfloat16(_EXP2_C))
        s_part = jnp.sum(e, axis=1, keepdims=True, dtype=jnp.bfloat16)
        m_s[...] = jnp.maximum(m_s[...], m_part)
        s_s[...] = s_s[...] + s_part.astype(jnp.float32)

    @pl.when(j == nj - 1)
    def _finish():
        out_ref[0] = m_s[...] * (1.0 / _P_SCALE) - jnp.log(s_s[...])


@functools.partial(jax.jit, static_argnames=("bn",))
def _argmax_logp(t0, t1, proto_t, bn):
    bsz, p1, k = t0.shape
    bm = p1 - 1
    n = proto_t.shape[1]
    grid = (2, bsz, n // bn)
    ni = bsz
    return pl.pallas_call(
        _lse_body,
        grid=grid,
        in_specs=[
            pl.BlockSpec((1, p1, k), lambda b, i, j: ((1 - b) * i + b * (ni - 1), 0, 0)),
            pl.BlockSpec((1, p1, k), lambda b, i, j: (b * i, 0, 0)),
            pl.BlockSpec((1, bn, k), lambda b, i, j: (b, j, 0)),
        ],
        out_specs=pl.BlockSpec((1, bm, 1), lambda b, i, j: (b, i, 0)),
        out_shape=jax.ShapeDtypeStruct((2, bsz * bm, 1), jnp.float32),
        scratch_shapes=[
            pltpu.VMEM((bm, k), proto_t.dtype),
            pltpu.VMEM((bm, 1), jnp.float32),
            pltpu.VMEM((bm, 1), jnp.float32),
        ],
        compiler_params=pltpu.CompilerParams(
            dimension_semantics=("parallel", "parallel", "arbitrary"),
        ),
    )(t0, t1, proto_t)


def kernel(img_tokens_0, img_tokens_1, prototypes, normal_mask):
    proto_t = prototypes.astype(jnp.float8_e4m3fn)
    bn = min(8192, prototypes.shape[1])
    vals = _argmax_logp(img_tokens_0, img_tokens_1, proto_t, bn)[..., 0]

    w = normal_mask.reshape(-1).astype(jnp.float32)
    ce = -(vals * w[None, :]).sum(axis=1) / w.sum()
    return ce.mean()
